# baseline (speedup 1.0000x reference)
import jax
import jax.numpy as jnp
from jax import lax
from jax.experimental import pallas as pl
from jax.experimental.pallas import tpu as pltpu
from jax.experimental.pallas import tpu_sc as plsc

N_USERS = 50000
N_ITEMS = 50000
N_NODES = N_USERS + N_ITEMS
NNZ = 1600000
D = 32

_SUPER = 512
_GB = 128
_NG = _SUPER // _GB
_EP = 200
_NPASS = 2
_PAD = 4 * _SUPER


def _build(n_users, nnz, super_, gb, ep, npass):
    d = D
    ng = super_ // gb
    lanes = 16
    n_sub = 16
    qrows = n_users // npass
    nblk = qrows // ep
    nb_per_tile = -(-nblk // n_sub)
    assert qrows % ep == 0 and ep % 8 == 0
    assert super_ % (8 * lanes) == 0 and (ep * d) % lanes == 0

    mesh = plsc.VectorSubcoreMesh(
        core_axis_name="c", subcore_axis_name="s", num_cores=2, num_subcores=n_sub
    )

    def body(ego, colp, rowp, valp, bounds, allout,
             bounds_v,
             colfA, rowfA, valfA, col2dA, lidxA, rowsA,
             colfB, rowfB, valfB, col2dB, lidxB, rowsB,
             acc_v, ego_v, out_v, accum,
             gsemA, gsemB, ssemA, ssemB):
        c = lax.axis_index("c").astype(jnp.int32)
        s = lax.axis_index("s").astype(jnp.int32)
        i16 = lax.broadcasted_iota(jnp.int32, (lanes,), 0)
        zero16 = jnp.zeros((lanes,), jnp.float32)

        pltpu.sync_copy(bounds.at[c], bounds_v)
        bv = bounds_v[...]

        cf = c.astype(jnp.float32)
        coef = 0.5 + cf

        bufA = (colfA, rowfA, valfA, col2dA, lidxA, rowsA, gsemA, ssemA)
        bufB = (colfB, rowfB, valfB, col2dB, lidxB, rowsB, gsemB, ssemB)

        for p in range(npass):
            lo_sc = bv[2 * p]
            hi_sc = bv[2 * p + 1]
            row_base = c * n_users + p * qrows

            cnt = hi_sc - lo_sc
            per = (cnt + (n_sub - 1)) // n_sub
            lo = lo_sc + jnp.minimum(s * per, cnt)
            hi = lo_sc + jnp.minimum((s + 1) * per, cnt)
            base = (lo // 8) * 8
            n_pairs = (hi - base + (2 * super_ - 1)) // (2 * super_)

            def load_prep_fire(start, buf):
                colf, rowf, valf, col2d, lidx2d, rows_buf, gsem, _ = buf
                pltpu.sync_copy(colp.at[pl.ds(start, super_)], colf)
                pltpu.sync_copy(rowp.at[pl.ds(start, super_)], rowf)
                pltpu.sync_copy(valp.at[pl.ds(start, super_)], valf)

                def prep(k, carry2):
                    for sub in range(super_ // lanes // ng):
                        off = sub * lanes
                        el = k * (super_ // ng) + off + i16
                        e = start + el
                        valid = (e >= lo) & (e < hi)
                        cidx = plsc.load_gather(colf, [el])
                        r = plsc.load_gather(rowf, [el])
                        v = plsc.load_gather(valf, [el])
                        lr = jnp.where(valid, r - row_base, 0)
                        v = jnp.where(valid, v, 0.0)
                        plsc.store_scatter(valf, [el], v)
                        col2d[k, pl.ds(off, lanes)] = cidx
                        lidx2d[k, pl.ds(off, lanes)] = lr
                    return carry2
                lax.fori_loop(0, ng, prep, 0)

                for k in range(ng):
                    pltpu.async_copy(ego.at[col2d.at[k]],
                                     rows_buf.at[pl.ds(k * gb, gb)], gsem)

            def drain_gather(buf):
                colf, rowf, valf, col2d, lidx2d, rows_buf, gsem, _ = buf
                for k in range(ng):
                    pltpu.make_async_copy(ego.at[col2d.at[k]],
                                          rows_buf.at[pl.ds(k * gb, gb)],
                                          gsem).wait()

            def scale_scatter(buf):
                colf, rowf, valf, col2d, lidx2d, rows_buf, _, ssem = buf

                def scale(j, carry2):
                    off = j * lanes
                    v16 = plsc.load_gather(valf, [off + i16])
                    for ee in range(lanes):
                        e = off + ee
                        v = v16[ee]
                        for h in range(d // lanes):
                            r = rows_buf[e, pl.ds(h * lanes, lanes)]
                            rows_buf[e, pl.ds(h * lanes, lanes)] = r * v
                    return carry2
                lax.fori_loop(0, super_ // lanes, scale, 0)

                puts = [pltpu.async_copy(rows_buf.at[pl.ds(k * gb, gb)],
                                         accum.at[lidx2d.at[k]], ssem, add=True)
                        for k in range(ng)]
                for dsc in puts:
                    dsc.wait()

            load_prep_fire(base, bufA)

            def zvreg(t, carry):
                acc_v[t, pl.ds(0, lanes)] = zero16
                acc_v[t, pl.ds(lanes, lanes)] = zero16
                return carry
            lax.fori_loop(0, ep, zvreg, 0)

            def zcopy(i, carry):
                bid = s + i * n_sub
                @pl.when(bid < nblk)
                def _():
                    pltpu.sync_copy(acc_v, accum.at[pl.ds(bid * ep, ep)])
                return carry
            lax.fori_loop(0, nb_per_tile, zcopy, 0)
            plsc.subcore_barrier()

            def pair(i, carry):
                startB = base + i * (2 * super_) + super_
                load_prep_fire(startB, bufB)
                drain_gather(bufA)
                scale_scatter(bufA)
                load_prep_fire(startB + super_, bufA)
                drain_gather(bufB)
                scale_scatter(bufB)
                return carry
            lax.fori_loop(0, n_pairs, pair, 0)
            drain_gather(bufA)
            plsc.subcore_barrier()

            def ep_blk(i, carry):
                bid = s + i * n_sub

                @pl.when(bid < nblk)
                def _():
                    r0 = bid * ep
                    pltpu.sync_copy(accum.at[pl.ds(r0, ep)], acc_v)
                    pltpu.sync_copy(ego.at[pl.ds(row_base + r0, ep)], ego_v)

                    def comp(t, carry2):
                        a0 = acc_v[t, pl.ds(0, lanes)]
                        a1 = acc_v[t, pl.ds(lanes, lanes)]
                        e0 = ego_v[t, pl.ds(0, lanes)]
                        e1 = ego_v[t, pl.ds(lanes, lanes)]
                        out_v[t, pl.ds(0, lanes)] = 0.5 * a0 + coef * e0
                        out_v[t, pl.ds(lanes, lanes)] = 0.5 * a1 + coef * e1
                        return carry2
                    lax.fori_loop(0, ep, comp, 0)

                    pltpu.sync_copy(out_v, allout.at[pl.ds(row_base + r0, ep)])
                return carry
            lax.fori_loop(0, nb_per_tile, ep_blk, 0)
            plsc.subcore_barrier()

    return pl.kernel(
        body,
        out_type=[
            jax.ShapeDtypeStruct((2 * n_users, d), jnp.float32),
        ],
        mesh=mesh,
        compiler_params=pltpu.CompilerParams(
            needs_layout_passes=False,
            use_tc_tiling_on_sc=False,
        ),
        scratch_types=[
            pltpu.VMEM((lanes,), jnp.int32),
            pltpu.VMEM((super_,), jnp.int32),
            pltpu.VMEM((super_,), jnp.int32),
            pltpu.VMEM((super_,), jnp.float32),
            pltpu.VMEM((ng, gb), jnp.int32),
            pltpu.VMEM((ng, gb), jnp.int32),
            pltpu.VMEM((super_, d), jnp.float32),
            pltpu.VMEM((super_,), jnp.int32),
            pltpu.VMEM((super_,), jnp.int32),
            pltpu.VMEM((super_,), jnp.float32),
            pltpu.VMEM((ng, gb), jnp.int32),
            pltpu.VMEM((ng, gb), jnp.int32),
            pltpu.VMEM((super_, d), jnp.float32),
            pltpu.VMEM((ep, d), jnp.float32),
            pltpu.VMEM((ep, d), jnp.float32),
            pltpu.VMEM((ep, d), jnp.float32),
            pltpu.VMEM_SHARED((qrows, d), jnp.float32),
            pltpu.SemaphoreType.DMA,
            pltpu.SemaphoreType.DMA,
            pltpu.SemaphoreType.DMA,
            pltpu.SemaphoreType.DMA,
        ],
    )


@jax.jit
def _run(user_emb, item_emb, adj_row, adj_col, adj_val):
    ego = jnp.concatenate([user_emb, item_emb], axis=0)
    row = adj_row.astype(jnp.int32)
    col = adj_col.astype(jnp.int32)
    val = adj_val.astype(jnp.float32)
    qrows = N_USERS // _NPASS
    cuts = jnp.arange(1, 2 * _NPASS, dtype=jnp.int32) * qrows
    bs = jnp.searchsorted(row, cuts, side="left").astype(jnp.int32)
    edges = jnp.concatenate([jnp.zeros((1,), jnp.int32), bs,
                             jnp.full((1,), NNZ, jnp.int32)])
    pairs = jnp.stack([edges[:-1], edges[1:]], axis=1).reshape(2, 2 * _NPASS)
    bounds = jnp.zeros((2, 16), jnp.int32).at[:, : 2 * _NPASS].set(pairs)
    zpad_i = jnp.zeros((_PAD,), jnp.int32)
    colp = jnp.concatenate([col, zpad_i])
    rowp = jnp.concatenate([row, zpad_i])
    valp = jnp.concatenate([val, jnp.zeros((_PAD,), jnp.float32)])
    (allout,) = _build(N_USERS, NNZ, _SUPER, _GB, _EP, _NPASS)(
        ego, colp, rowp, valp, bounds)
    return (allout[:N_USERS], allout[N_USERS:])


def kernel(user_emb, item_emb, adj_row, adj_col, adj_val):
    return _run(user_emb, item_emb, adj_row, adj_col, adj_val)


# speedup vs baseline: 1.0559x; 1.0559x over previous
import jax
import jax.numpy as jnp
from jax import lax
from jax.experimental import pallas as pl
from jax.experimental.pallas import tpu as pltpu
from jax.experimental.pallas import tpu_sc as plsc

N_USERS = 50000
N_ITEMS = 50000
N_NODES = N_USERS + N_ITEMS
NNZ = 1600000
D = 32

_SUPER = 1024
_GB = 128
_NG = _SUPER // _GB
_EP = 200
_NPASS = 2
_PAD = 2 * _SUPER


def _build(n_users, nnz, super_, gb, ep, npass):
    d = D
    ng = super_ // gb
    lanes = 16
    n_sub = 16
    qrows = n_users // npass
    nblk = qrows // ep
    nb_per_tile = -(-nblk // n_sub)
    assert qrows % ep == 0 and ep % 8 == 0
    assert super_ % (8 * lanes) == 0 and (ep * d) % lanes == 0

    mesh = plsc.VectorSubcoreMesh(
        core_axis_name="c", subcore_axis_name="s", num_cores=2, num_subcores=n_sub
    )

    def body(ego, colp, rowp, valp, bounds, allout,
             bounds_v,
             colfA, rowfA, valfA, col2dA, lidxA, rowsA,
             acc_v, ego_v, out_v, accum,
             gsemA, ssemA):
        c = lax.axis_index("c").astype(jnp.int32)
        s = lax.axis_index("s").astype(jnp.int32)
        i16 = lax.broadcasted_iota(jnp.int32, (lanes,), 0)
        zero16 = jnp.zeros((lanes,), jnp.float32)

        pltpu.sync_copy(bounds.at[c], bounds_v)
        bv = bounds_v[...]

        cf = c.astype(jnp.float32)
        coef = 0.5 + cf

        buf = (colfA, rowfA, valfA, col2dA, lidxA, rowsA, gsemA, ssemA)

        for p in range(npass):
            lo_sc = bv[2 * p]
            hi_sc = bv[2 * p + 1]
            row_base = c * n_users + p * qrows

            cnt = hi_sc - lo_sc
            per = (cnt + (n_sub - 1)) // n_sub
            lo = lo_sc + jnp.minimum(s * per, cnt)
            hi = lo_sc + jnp.minimum((s + 1) * per, cnt)
            base = (lo // 8) * 8
            n_chunks = (hi - base + (super_ - 1)) // super_

            def zvreg(t, carry):
                acc_v[t, pl.ds(0, lanes)] = zero16
                acc_v[t, pl.ds(lanes, lanes)] = zero16
                return carry
            lax.fori_loop(0, ep, zvreg, 0)

            def zcopy(i, carry):
                bid = s + i * n_sub
                @pl.when(bid < nblk)
                def _():
                    pltpu.sync_copy(acc_v, accum.at[pl.ds(bid * ep, ep)])
                return carry
            lax.fori_loop(0, nb_per_tile, zcopy, 0)
            plsc.subcore_barrier()

            def chunk(g, carry):
                colf, rowf, valf, col2d, lidx2d, rows_buf, gsem, ssem = buf
                start = base + g * super_
                lin = [pltpu.async_copy(colp.at[pl.ds(start, super_)], colf, gsem),
                       pltpu.async_copy(rowp.at[pl.ds(start, super_)], rowf, gsem),
                       pltpu.async_copy(valp.at[pl.ds(start, super_)], valf, gsem)]
                for dsc in lin:
                    dsc.wait()

                def prep(k, carry2):
                    for sub in range(super_ // lanes // ng):
                        off = sub * lanes
                        el = k * (super_ // ng) + off + i16
                        e = start + el
                        valid = (e >= lo) & (e < hi)
                        cidx = plsc.load_gather(colf, [el])
                        r = plsc.load_gather(rowf, [el])
                        v = plsc.load_gather(valf, [el])
                        lr = jnp.where(valid, r - row_base, 0)
                        v = jnp.where(valid, v, 0.0)
                        plsc.store_scatter(valf, [el], v)
                        col2d[k, pl.ds(off, lanes)] = cidx
                        lidx2d[k, pl.ds(off, lanes)] = lr
                    return carry2
                lax.fori_loop(0, ng, prep, 0)

                gets = [pltpu.async_copy(ego.at[col2d.at[k]],
                                         rows_buf.at[pl.ds(k * gb, gb)], gsem)
                        for k in range(ng)]
                for dsc in gets:
                    dsc.wait()

                def scale(j, carry2):
                    off = j * lanes
                    v16 = plsc.load_gather(valf, [off + i16])
                    for ee in range(lanes):
                        e = off + ee
                        v = v16[ee]
                        for h in range(d // lanes):
                            r = rows_buf[e, pl.ds(h * lanes, lanes)]
                            rows_buf[e, pl.ds(h * lanes, lanes)] = r * v
                    return carry2
                lax.fori_loop(0, super_ // lanes, scale, 0)

                puts = [pltpu.async_copy(rows_buf.at[pl.ds(k * gb, gb)],
                                         accum.at[lidx2d.at[k]], ssem, add=True)
                        for k in range(ng)]
                for dsc in puts:
                    dsc.wait()
                return carry
            lax.fori_loop(0, n_chunks, chunk, 0)
            plsc.subcore_barrier()

            def ep_blk(i, carry):
                bid = s + i * n_sub

                @pl.when(bid < nblk)
                def _():
                    r0 = bid * ep
                    pltpu.sync_copy(accum.at[pl.ds(r0, ep)], acc_v)
                    pltpu.sync_copy(ego.at[pl.ds(row_base + r0, ep)], ego_v)

                    def comp(t, carry2):
                        a0 = acc_v[t, pl.ds(0, lanes)]
                        a1 = acc_v[t, pl.ds(lanes, lanes)]
                        e0 = ego_v[t, pl.ds(0, lanes)]
                        e1 = ego_v[t, pl.ds(lanes, lanes)]
                        out_v[t, pl.ds(0, lanes)] = 0.5 * a0 + coef * e0
                        out_v[t, pl.ds(lanes, lanes)] = 0.5 * a1 + coef * e1
                        return carry2
                    lax.fori_loop(0, ep, comp, 0)

                    pltpu.sync_copy(out_v, allout.at[pl.ds(row_base + r0, ep)])
                return carry
            lax.fori_loop(0, nb_per_tile, ep_blk, 0)
            plsc.subcore_barrier()

    return pl.kernel(
        body,
        out_type=[
            jax.ShapeDtypeStruct((2 * n_users, d), jnp.float32),
        ],
        mesh=mesh,
        compiler_params=pltpu.CompilerParams(
            needs_layout_passes=False,
            use_tc_tiling_on_sc=False,
        ),
        scratch_types=[
            pltpu.VMEM((lanes,), jnp.int32),
            pltpu.VMEM((super_,), jnp.int32),
            pltpu.VMEM((super_,), jnp.int32),
            pltpu.VMEM((super_,), jnp.float32),
            pltpu.VMEM((ng, gb), jnp.int32),
            pltpu.VMEM((ng, gb), jnp.int32),
            pltpu.VMEM((super_, d), jnp.float32),
            pltpu.VMEM((ep, d), jnp.float32),
            pltpu.VMEM((ep, d), jnp.float32),
            pltpu.VMEM((ep, d), jnp.float32),
            pltpu.VMEM_SHARED((qrows, d), jnp.float32),
            pltpu.SemaphoreType.DMA,
            pltpu.SemaphoreType.DMA,
        ],
    )


@jax.jit
def _run(user_emb, item_emb, adj_row, adj_col, adj_val):
    ego = jnp.concatenate([user_emb, item_emb], axis=0)
    row = adj_row.astype(jnp.int32)
    col = adj_col.astype(jnp.int32)
    val = adj_val.astype(jnp.float32)
    qrows = N_USERS // _NPASS
    cuts = jnp.arange(1, 2 * _NPASS, dtype=jnp.int32) * qrows
    bs = jnp.searchsorted(row, cuts, side="left").astype(jnp.int32)
    edges = jnp.concatenate([jnp.zeros((1,), jnp.int32), bs,
                             jnp.full((1,), NNZ, jnp.int32)])
    pairs = jnp.stack([edges[:-1], edges[1:]], axis=1).reshape(2, 2 * _NPASS)
    bounds = jnp.zeros((2, 16), jnp.int32).at[:, : 2 * _NPASS].set(pairs)
    zpad_i = jnp.zeros((_PAD,), jnp.int32)
    colp = jnp.concatenate([col, zpad_i])
    rowp = jnp.concatenate([row, zpad_i])
    valp = jnp.concatenate([val, jnp.zeros((_PAD,), jnp.float32)])
    (allout,) = _build(N_USERS, NNZ, _SUPER, _GB, _EP, _NPASS)(
        ego, colp, rowp, valp, bounds)
    return (allout[:N_USERS], allout[N_USERS:])


def kernel(user_emb, item_emb, adj_row, adj_col, adj_val):
    return _run(user_emb, item_emb, adj_row, adj_col, adj_val)


# speedup vs baseline: 1.1070x; 1.0483x over previous
import jax
import jax.numpy as jnp
from jax import lax
from jax.experimental import pallas as pl
from jax.experimental.pallas import tpu as pltpu
from jax.experimental.pallas import tpu_sc as plsc

N_USERS = 50000
N_ITEMS = 50000
N_NODES = N_USERS + N_ITEMS
NNZ = 1600000
D = 32

_SUPER = 1024
_GB = 128
_NG = _SUPER // _GB
_EP = 200
_NPASS = 2
_PAD = 2 * _SUPER


def _build(n_users, nnz, super_, gb, ep, npass):
    d = D
    ng = super_ // gb
    lanes = 16
    n_sub = 16
    qrows = n_users // npass
    nblk = qrows // ep
    nb_per_tile = -(-nblk // n_sub)
    assert qrows % ep == 0 and ep % 8 == 0
    assert super_ % (8 * lanes) == 0 and (ep * d) % lanes == 0

    mesh = plsc.VectorSubcoreMesh(
        core_axis_name="c", subcore_axis_name="s", num_cores=2, num_subcores=n_sub
    )

    def body(ego, colp, rowp, valp, bounds, allout,
             bounds_v,
             colfA, rowfA, valfA, valmA, col2dA, lidxA, rowsA,
             acc_v, ego_v, out_v, accum,
             gsemA, ssemA, lsemA):
        c = lax.axis_index("c").astype(jnp.int32)
        s = lax.axis_index("s").astype(jnp.int32)
        i16 = lax.broadcasted_iota(jnp.int32, (lanes,), 0)
        zero16 = jnp.zeros((lanes,), jnp.float32)

        pltpu.sync_copy(bounds.at[c], bounds_v)
        bv = bounds_v[...]

        cf = c.astype(jnp.float32)
        coef = 0.5 + cf


        for p in range(npass):
            lo_sc = bv[2 * p]
            hi_sc = bv[2 * p + 1]
            row_base = c * n_users + p * qrows

            cnt = hi_sc - lo_sc
            per = (cnt + (n_sub - 1)) // n_sub
            lo = lo_sc + jnp.minimum(s * per, cnt)
            hi = lo_sc + jnp.minimum((s + 1) * per, cnt)
            base = (lo // 8) * 8
            n_chunks = (hi - base + (super_ - 1)) // super_

            def zvreg(t, carry):
                acc_v[t, pl.ds(0, lanes)] = zero16
                acc_v[t, pl.ds(lanes, lanes)] = zero16
                return carry
            lax.fori_loop(0, ep, zvreg, 0)

            def zcopy(i, carry):
                bid = s + i * n_sub
                @pl.when(bid < nblk)
                def _():
                    pltpu.sync_copy(acc_v, accum.at[pl.ds(bid * ep, ep)])
                return carry
            lax.fori_loop(0, nb_per_tile, zcopy, 0)
            plsc.subcore_barrier()

            pltpu.async_copy(colp.at[pl.ds(base, super_)], colfA, lsemA)
            pltpu.async_copy(rowp.at[pl.ds(base, super_)], rowfA, lsemA)
            pltpu.async_copy(valp.at[pl.ds(base, super_)], valfA, lsemA)

            def chunk(g, carry):
                start = base + g * super_
                pltpu.make_async_copy(colp.at[pl.ds(start, super_)], colfA, lsemA).wait()
                pltpu.make_async_copy(rowp.at[pl.ds(start, super_)], rowfA, lsemA).wait()
                pltpu.make_async_copy(valp.at[pl.ds(start, super_)], valfA, lsemA).wait()

                def prep(k, carry2):
                    for sub in range(super_ // lanes // ng):
                        off = sub * lanes
                        el = k * (super_ // ng) + off + i16
                        e = start + el
                        valid = (e >= lo) & (e < hi)
                        cidx = plsc.load_gather(colfA, [el])
                        r = plsc.load_gather(rowfA, [el])
                        v = plsc.load_gather(valfA, [el])
                        lr = jnp.where(valid, r - row_base, 0)
                        v = jnp.where(valid, v, 0.0)
                        plsc.store_scatter(valmA, [el], v)
                        col2dA[k, pl.ds(off, lanes)] = cidx
                        lidxA[k, pl.ds(off, lanes)] = lr
                    return carry2
                lax.fori_loop(0, ng, prep, 0)

                nstart = start + super_
                pltpu.async_copy(colp.at[pl.ds(nstart, super_)], colfA, lsemA)
                pltpu.async_copy(rowp.at[pl.ds(nstart, super_)], rowfA, lsemA)
                pltpu.async_copy(valp.at[pl.ds(nstart, super_)], valfA, lsemA)

                gets = [pltpu.async_copy(ego.at[col2dA.at[k]],
                                         rowsA.at[pl.ds(k * gb, gb)], gsemA)
                        for k in range(ng)]
                for dsc in gets:
                    dsc.wait()

                def scale(j, carry2):
                    off = j * lanes
                    v16 = plsc.load_gather(valmA, [off + i16])
                    for ee in range(lanes):
                        e = off + ee
                        v = v16[ee]
                        for h in range(d // lanes):
                            r = rowsA[e, pl.ds(h * lanes, lanes)]
                            rowsA[e, pl.ds(h * lanes, lanes)] = r * v
                    return carry2
                lax.fori_loop(0, super_ // lanes, scale, 0)

                puts = [pltpu.async_copy(rowsA.at[pl.ds(k * gb, gb)],
                                         accum.at[lidxA.at[k]], ssemA, add=True)
                        for k in range(ng)]
                for dsc in puts:
                    dsc.wait()
                return carry
            lax.fori_loop(0, n_chunks, chunk, 0)
            pltpu.make_async_copy(colp.at[pl.ds(base, super_)], colfA, lsemA).wait()
            pltpu.make_async_copy(rowp.at[pl.ds(base, super_)], rowfA, lsemA).wait()
            pltpu.make_async_copy(valp.at[pl.ds(base, super_)], valfA, lsemA).wait()
            plsc.subcore_barrier()

            def ep_blk(i, carry):
                bid = s + i * n_sub

                @pl.when(bid < nblk)
                def _():
                    r0 = bid * ep
                    pltpu.sync_copy(accum.at[pl.ds(r0, ep)], acc_v)
                    pltpu.sync_copy(ego.at[pl.ds(row_base + r0, ep)], ego_v)

                    def comp(t, carry2):
                        a0 = acc_v[t, pl.ds(0, lanes)]
                        a1 = acc_v[t, pl.ds(lanes, lanes)]
                        e0 = ego_v[t, pl.ds(0, lanes)]
                        e1 = ego_v[t, pl.ds(lanes, lanes)]
                        out_v[t, pl.ds(0, lanes)] = 0.5 * a0 + coef * e0
                        out_v[t, pl.ds(lanes, lanes)] = 0.5 * a1 + coef * e1
                        return carry2
                    lax.fori_loop(0, ep, comp, 0)

                    pltpu.sync_copy(out_v, allout.at[pl.ds(row_base + r0, ep)])
                return carry
            lax.fori_loop(0, nb_per_tile, ep_blk, 0)
            plsc.subcore_barrier()

    return pl.kernel(
        body,
        out_type=[
            jax.ShapeDtypeStruct((2 * n_users, d), jnp.float32),
        ],
        mesh=mesh,
        compiler_params=pltpu.CompilerParams(
            needs_layout_passes=False,
            use_tc_tiling_on_sc=False,
        ),
        scratch_types=[
            pltpu.VMEM((lanes,), jnp.int32),
            pltpu.VMEM((super_,), jnp.int32),
            pltpu.VMEM((super_,), jnp.int32),
            pltpu.VMEM((super_,), jnp.float32),
            pltpu.VMEM((super_,), jnp.float32),
            pltpu.VMEM((ng, gb), jnp.int32),
            pltpu.VMEM((ng, gb), jnp.int32),
            pltpu.VMEM((super_, d), jnp.float32),
            pltpu.VMEM((ep, d), jnp.float32),
            pltpu.VMEM((ep, d), jnp.float32),
            pltpu.VMEM((ep, d), jnp.float32),
            pltpu.VMEM_SHARED((qrows, d), jnp.float32),
            pltpu.SemaphoreType.DMA,
            pltpu.SemaphoreType.DMA,
            pltpu.SemaphoreType.DMA,
        ],
    )


@jax.jit
def _run(user_emb, item_emb, adj_row, adj_col, adj_val):
    ego = jnp.concatenate([user_emb, item_emb], axis=0)
    row = adj_row.astype(jnp.int32)
    col = adj_col.astype(jnp.int32)
    val = adj_val.astype(jnp.float32)
    qrows = N_USERS // _NPASS
    cuts = jnp.arange(1, 2 * _NPASS, dtype=jnp.int32) * qrows
    bs = jnp.searchsorted(row, cuts, side="left").astype(jnp.int32)
    edges = jnp.concatenate([jnp.zeros((1,), jnp.int32), bs,
                             jnp.full((1,), NNZ, jnp.int32)])
    pairs = jnp.stack([edges[:-1], edges[1:]], axis=1).reshape(2, 2 * _NPASS)
    bounds = jnp.zeros((2, 16), jnp.int32).at[:, : 2 * _NPASS].set(pairs)
    zpad_i = jnp.zeros((_PAD,), jnp.int32)
    colp = jnp.concatenate([col, zpad_i])
    rowp = jnp.concatenate([row, zpad_i])
    valp = jnp.concatenate([val, jnp.zeros((_PAD,), jnp.float32)])
    (allout,) = _build(N_USERS, NNZ, _SUPER, _GB, _EP, _NPASS)(
        ego, colp, rowp, valp, bounds)
    return (allout[:N_USERS], allout[N_USERS:])


def kernel(user_emb, item_emb, adj_row, adj_col, adj_val):
    return _run(user_emb, item_emb, adj_row, adj_col, adj_val)
